# baseline (speedup 1.0000x reference)
import functools

import jax
import jax.numpy as jnp
from jax import lax
from jax.experimental import pallas as pl
from jax.experimental.pallas import tpu as pltpu
from jax.experimental.pallas import tpu_sc as plsc

N = 4096
LANES = 16
NUM_CORES = 2
NUM_SUBCORES = 16
NW = NUM_CORES * NUM_SUBCORES
ROWS_PER_W = N // NW
CH_ROWS = 8
NCHUNK = ROWS_PER_W // CH_ROWS
ROW_SLICES = N // LANES

UNROLL = 16


@functools.partial(
    pl.kernel,
    mesh=plsc.VectorSubcoreMesh(core_axis_name="c", subcore_axis_name="s"),
    out_type=jax.ShapeDtypeStruct((NW, LANES), jnp.int32),
    scratch_types=[
        pltpu.VMEM((CH_ROWS, N), jnp.float32),
        pltpu.VMEM((CH_ROWS, N), jnp.float32),
        pltpu.VMEM((LANES,), jnp.int32),
        pltpu.SemaphoreType.DMA,
        pltpu.SemaphoreType.DMA,
    ],
)
def _roundtrip_check(x_hbm, out_hbm, buf0, buf1, violbuf, sem0, sem1):
    wid = lax.axis_index("s") * NUM_CORES + lax.axis_index("c")
    base = wid * ROWS_PER_W
    bufs = (buf0, buf1)
    sems = (sem0, sem1)

    pend = pltpu.async_copy(x_hbm.at[pl.ds(base, CH_ROWS)], bufs[0], sems[0])
    amax = jnp.zeros((LANES,), jnp.int32)
    for c in range(NCHUNK):
        pend.wait()
        if c + 1 < NCHUNK:
            pend = pltpu.async_copy(
                x_hbm.at[pl.ds(base + (c + 1) * CH_ROWS, CH_ROWS)],
                bufs[(c + 1) % 2], sems[(c + 1) % 2])
        buf = bufs[c % 2]

        for r in range(CH_ROWS):

            def body(i, acc, buf=buf, r=r):
                off = i * (LANES * UNROLL)
                for u in range(UNROLL):
                    v = buf[r, pl.ds(off + u * LANES, LANES)]
                    w = lax.bitcast_convert_type(v, jnp.int32)
                    acc = jnp.maximum(acc, w & 0x7FFFFFFF)
                return acc

            amax = lax.fori_loop(0, ROW_SLICES // UNROLL, body, amax)

    violbuf[...] = jnp.where(amax > 0x7F800000, 1, 0).astype(jnp.int32)
    pltpu.sync_copy(violbuf, out_hbm.at[wid])


def kernel(x):
    counts = _roundtrip_check(x)
    return (jnp.sum(counts) == 0).reshape(1)


# speedup vs baseline: 1.3536x; 1.3536x over previous
import functools

import jax
import jax.numpy as jnp
from jax import lax
from jax.experimental import pallas as pl
from jax.experimental.pallas import tpu as pltpu
from jax.experimental.pallas import tpu_sc as plsc

N = 4096
LANES = 16
NUM_CORES = 2
NUM_SUBCORES = 16
NW = NUM_CORES * NUM_SUBCORES

TC_ROWS = 2048
SC_ROWS = N - TC_ROWS
SC_BASE = TC_ROWS

ROWS_PER_W = SC_ROWS // NW
CH_ROWS = 8
NCHUNK = ROWS_PER_W // CH_ROWS
ROW_SLICES = N // LANES
UNROLL = 16

TC_BLOCK_ROWS = 256

_EXP_MASK = 0x7F800000
_ABS_MASK = 0x7FFFFFFF


@functools.partial(
    pl.kernel,
    mesh=plsc.VectorSubcoreMesh(core_axis_name="c", subcore_axis_name="s"),
    out_type=jax.ShapeDtypeStruct((NW, LANES), jnp.int32),
    scratch_types=[
        pltpu.VMEM((CH_ROWS, N), jnp.float32),
        pltpu.VMEM((CH_ROWS, N), jnp.float32),
        pltpu.VMEM((LANES,), jnp.int32),
        pltpu.SemaphoreType.DMA,
        pltpu.SemaphoreType.DMA,
    ],
)
def _sc_check(x_hbm, out_hbm, buf0, buf1, violbuf, sem0, sem1):
    wid = lax.axis_index("s") * NUM_CORES + lax.axis_index("c")
    base = SC_BASE + wid * ROWS_PER_W
    bufs = (buf0, buf1)
    sems = (sem0, sem1)

    pend = pltpu.async_copy(x_hbm.at[pl.ds(base, CH_ROWS)], bufs[0], sems[0])
    amax = jnp.zeros((LANES,), jnp.int32)
    for c in range(NCHUNK):
        pend.wait()
        if c + 1 < NCHUNK:
            pend = pltpu.async_copy(
                x_hbm.at[pl.ds(base + (c + 1) * CH_ROWS, CH_ROWS)],
                bufs[(c + 1) % 2], sems[(c + 1) % 2])
        buf = bufs[c % 2]

        for r in range(CH_ROWS):

            def body(i, acc, buf=buf, r=r):
                off = i * (LANES * UNROLL)
                for u in range(UNROLL):
                    v = buf[r, pl.ds(off + u * LANES, LANES)]
                    w = lax.bitcast_convert_type(v, jnp.int32)
                    acc = jnp.maximum(acc, w & _ABS_MASK)
                return acc

            amax = lax.fori_loop(0, ROW_SLICES // UNROLL, body, amax)

    violbuf[...] = jnp.where(amax > _EXP_MASK, 1, 0).astype(jnp.int32)
    pltpu.sync_copy(violbuf, out_hbm.at[wid])


def _tc_body(x_ref, out_ref):
    w = lax.bitcast_convert_type(x_ref[...], jnp.int32)
    nan_cnt = jnp.sum((((w & _ABS_MASK) > _EXP_MASK)).astype(jnp.int32))

    @pl.when(pl.program_id(0) == 0)
    def _():
        out_ref[0, 0] = 0

    out_ref[0, 0] += nan_cnt


_tc_check = pl.pallas_call(
    _tc_body,
    grid=(TC_ROWS // TC_BLOCK_ROWS,),
    in_specs=[pl.BlockSpec((TC_BLOCK_ROWS, N), lambda i: (i, 0))],
    out_specs=pl.BlockSpec(memory_space=pltpu.SMEM),
    out_shape=jax.ShapeDtypeStruct((1, 1), jnp.int32),
)


def kernel(x):
    sc_counts = _sc_check(x)
    tc_count = _tc_check(x)
    total = tc_count[0, 0] + jnp.sum(sc_counts)
    return (total == 0).reshape(1)


# speedup vs baseline: 1.4641x; 1.0816x over previous
import functools

import jax
import jax.numpy as jnp
from jax import lax
from jax.experimental import pallas as pl
from jax.experimental.pallas import tpu as pltpu
from jax.experimental.pallas import tpu_sc as plsc

N = 4096
LANES = 16
NUM_CORES = 2
NUM_SUBCORES = 16
NW = NUM_CORES * NUM_SUBCORES

TC_ROWS = 2560
SC_ROWS = N - TC_ROWS
SC_BASE = TC_ROWS

ROWS_PER_W = SC_ROWS // NW
CH_ROWS = 8
NCHUNK = ROWS_PER_W // CH_ROWS
ROW_SLICES = N // LANES
UNROLL = 16

TC_BLOCK_ROWS = 256

_EXP_MASK = 0x7F800000
_ABS_MASK = 0x7FFFFFFF


@functools.partial(
    pl.kernel,
    mesh=plsc.VectorSubcoreMesh(core_axis_name="c", subcore_axis_name="s"),
    out_type=jax.ShapeDtypeStruct((NW, LANES), jnp.int32),
    scratch_types=[
        pltpu.VMEM((CH_ROWS, N), jnp.float32),
        pltpu.VMEM((CH_ROWS, N), jnp.float32),
        pltpu.VMEM((LANES,), jnp.int32),
        pltpu.SemaphoreType.DMA,
        pltpu.SemaphoreType.DMA,
    ],
)
def _sc_check(x_hbm, out_hbm, buf0, buf1, violbuf, sem0, sem1):
    wid = lax.axis_index("s") * NUM_CORES + lax.axis_index("c")
    base = SC_BASE + wid * ROWS_PER_W
    bufs = (buf0, buf1)
    sems = (sem0, sem1)

    pend = pltpu.async_copy(x_hbm.at[pl.ds(base, CH_ROWS)], bufs[0], sems[0])
    amax = jnp.zeros((LANES,), jnp.int32)
    for c in range(NCHUNK):
        pend.wait()
        if c + 1 < NCHUNK:
            pend = pltpu.async_copy(
                x_hbm.at[pl.ds(base + (c + 1) * CH_ROWS, CH_ROWS)],
                bufs[(c + 1) % 2], sems[(c + 1) % 2])
        buf = bufs[c % 2]

        for r in range(CH_ROWS):

            def body(i, acc, buf=buf, r=r):
                off = i * (LANES * UNROLL)
                for u in range(UNROLL):
                    v = buf[r, pl.ds(off + u * LANES, LANES)]
                    w = lax.bitcast_convert_type(v, jnp.int32)
                    acc = jnp.maximum(acc, w & _ABS_MASK)
                return acc

            amax = lax.fori_loop(0, ROW_SLICES // UNROLL, body, amax)

    violbuf[...] = jnp.where(amax > _EXP_MASK, 1, 0).astype(jnp.int32)
    pltpu.sync_copy(violbuf, out_hbm.at[wid])


def _tc_body(x_ref, out_ref):
    w = lax.bitcast_convert_type(x_ref[...], jnp.int32)
    nan_cnt = jnp.sum((((w & _ABS_MASK) > _EXP_MASK)).astype(jnp.int32))

    @pl.when(pl.program_id(0) == 0)
    def _():
        out_ref[0, 0] = 0

    out_ref[0, 0] += nan_cnt


_tc_check = pl.pallas_call(
    _tc_body,
    grid=(TC_ROWS // TC_BLOCK_ROWS,),
    in_specs=[pl.BlockSpec((TC_BLOCK_ROWS, N), lambda i: (i, 0))],
    out_specs=pl.BlockSpec(memory_space=pltpu.SMEM),
    out_shape=jax.ShapeDtypeStruct((1, 1), jnp.int32),
)


def kernel(x):
    sc_counts = _sc_check(x)
    tc_count = _tc_check(x)
    total = tc_count[0, 0] + jnp.sum(sc_counts)
    return (total == 0).reshape(1)


# speedup vs baseline: 1.4846x; 1.0140x over previous
import functools

import jax
import jax.numpy as jnp
from jax import lax
from jax.experimental import pallas as pl
from jax.experimental.pallas import tpu as pltpu
from jax.experimental.pallas import tpu_sc as plsc

N = 4096
LANES = 16
NUM_CORES = 2
NUM_SUBCORES = 16
NW = NUM_CORES * NUM_SUBCORES

TC_ROWS = 3072
SC_ROWS = N - TC_ROWS
SC_BASE = TC_ROWS

ROWS_PER_W = SC_ROWS // NW
CH_ROWS = 8
NCHUNK = ROWS_PER_W // CH_ROWS
ROW_SLICES = N // LANES
UNROLL = 16

TC_BLOCK_ROWS = 256

_EXP_MASK = 0x7F800000
_ABS_MASK = 0x7FFFFFFF


@functools.partial(
    pl.kernel,
    mesh=plsc.VectorSubcoreMesh(core_axis_name="c", subcore_axis_name="s"),
    out_type=jax.ShapeDtypeStruct((NW, LANES), jnp.int32),
    scratch_types=[
        pltpu.VMEM((CH_ROWS, N), jnp.float32),
        pltpu.VMEM((CH_ROWS, N), jnp.float32),
        pltpu.VMEM((LANES,), jnp.int32),
        pltpu.SemaphoreType.DMA,
        pltpu.SemaphoreType.DMA,
    ],
)
def _sc_check(x_hbm, out_hbm, buf0, buf1, violbuf, sem0, sem1):
    wid = lax.axis_index("s") * NUM_CORES + lax.axis_index("c")
    base = SC_BASE + wid * ROWS_PER_W
    bufs = (buf0, buf1)
    sems = (sem0, sem1)

    pend = pltpu.async_copy(x_hbm.at[pl.ds(base, CH_ROWS)], bufs[0], sems[0])
    amax = jnp.zeros((LANES,), jnp.int32)
    for c in range(NCHUNK):
        pend.wait()
        if c + 1 < NCHUNK:
            pend = pltpu.async_copy(
                x_hbm.at[pl.ds(base + (c + 1) * CH_ROWS, CH_ROWS)],
                bufs[(c + 1) % 2], sems[(c + 1) % 2])
        buf = bufs[c % 2]

        for r in range(CH_ROWS):

            def body(i, acc, buf=buf, r=r):
                off = i * (LANES * UNROLL)
                for u in range(UNROLL):
                    v = buf[r, pl.ds(off + u * LANES, LANES)]
                    w = lax.bitcast_convert_type(v, jnp.int32)
                    acc = jnp.maximum(acc, w & _ABS_MASK)
                return acc

            amax = lax.fori_loop(0, ROW_SLICES // UNROLL, body, amax)

    violbuf[...] = jnp.where(amax > _EXP_MASK, 1, 0).astype(jnp.int32)
    pltpu.sync_copy(violbuf, out_hbm.at[wid])


def _tc_body(x_ref, out_ref):
    w = lax.bitcast_convert_type(x_ref[...], jnp.int32)
    nan_cnt = jnp.sum((((w & _ABS_MASK) > _EXP_MASK)).astype(jnp.int32))

    @pl.when(pl.program_id(0) == 0)
    def _():
        out_ref[0, 0] = 0

    out_ref[0, 0] += nan_cnt


_tc_check = pl.pallas_call(
    _tc_body,
    grid=(TC_ROWS // TC_BLOCK_ROWS,),
    in_specs=[pl.BlockSpec((TC_BLOCK_ROWS, N), lambda i: (i, 0))],
    out_specs=pl.BlockSpec(memory_space=pltpu.SMEM),
    out_shape=jax.ShapeDtypeStruct((1, 1), jnp.int32),
)


def kernel(x):
    sc_counts = _sc_check(x)
    tc_count = _tc_check(x)
    total = tc_count[0, 0] + jnp.sum(sc_counts)
    return (total == 0).reshape(1)
